# baseline (speedup 1.0000x reference)
import functools

import jax
import jax.numpy as jnp
from jax import lax
from jax.experimental import pallas as pl
from jax.experimental.pallas import tpu as pltpu
from jax.experimental.pallas import tpu_sc as plsc

N = 10000
E = 320000
G = 64
LATENT = 128
MAX_LV = 3

NC = 2
NS = 16
NW = NC * NS
CH = 64
K = 158
E_PAD = NW * K * CH
NP = 10112
RPS = NP // NS

NBUF = 4

_mesh = plsc.VectorSubcoreMesh(core_axis_name="c", subcore_axis_name="s")


def _edge_pool_body(ef_hbm, dst_hbm, z_hbm, out_hbm, dstv, rows, acc, sem):
    c = lax.axis_index("c")
    s = lax.axis_index("s")
    w = c * NS + s
    pltpu.sync_copy(z_hbm, acc.at[pl.ds(s * RPS, RPS)])
    plsc.subcore_barrier()

    def fetch(j, b):
        base = (w * K + j) * CH
        pltpu.sync_copy(dst_hbm.at[w, j], dstv.at[b])
        pltpu.async_copy(ef_hbm.at[pl.ds(base, CH)], rows.at[b], sem.at[b])

    fetch(0, 0)

    @pl.loop(0, K)
    def _chunk(j):
        b = lax.rem(j, 2)

        @pl.when(j + 1 < K)
        def _():
            fetch(j + 1, 1 - b)

        base = (w * K + j) * CH
        pltpu.make_async_copy(ef_hbm.at[pl.ds(base, CH)], rows.at[b],
                              sem.at[b]).wait()
        pltpu.sync_copy(rows.at[b], acc.at[dstv.at[b]], add=True)

    plsc.subcore_barrier()
    pltpu.sync_copy(acc.at[pl.ds(s * RPS, RPS)],
                    out_hbm.at[c, pl.ds(s * RPS, RPS)])


@functools.partial(
    pl.kernel,
    out_type=jax.ShapeDtypeStruct((NC, NP, LATENT), jnp.float32),
    mesh=_mesh,
    scratch_types=[
        pltpu.VMEM((2, CH), jnp.int32),
        pltpu.VMEM((2, CH, LATENT), jnp.float32),
        pltpu.VMEM_SHARED((NP, LATENT), jnp.float32),
        pltpu.SemaphoreType.DMA((2,)),
    ],
)
def _edge_pool(ef_hbm, dst_hbm, z_hbm, out_hbm, dstv, rows, acc, sem):
    _edge_pool_body(ef_hbm, dst_hbm, z_hbm, out_hbm, dstv, rows, acc, sem)


def _spmm_body(t_hbm, src_hbm, dst_hbm, z_hbm, out_hbm,
               srcv, dstv, rows, acc, sem):
    c = lax.axis_index("c")
    s = lax.axis_index("s")
    w = c * NS + s
    pltpu.sync_copy(z_hbm, acc.at[pl.ds(s * RPS, RPS)])
    plsc.subcore_barrier()

    def fetch(j, b):
        pltpu.sync_copy(src_hbm.at[w, j], srcv.at[b])
        pltpu.sync_copy(dst_hbm.at[w, j], dstv.at[b])
        pltpu.async_copy(t_hbm.at[srcv.at[b]], rows.at[b], sem.at[b])

    for b0 in range(NBUF):
        fetch(b0, b0)

    @pl.loop(0, K)
    def _chunk(j):
        b = lax.rem(j, NBUF)

        pltpu.make_async_copy(t_hbm.at[srcv.at[b]], rows.at[b],
                              sem.at[b]).wait()
        pltpu.sync_copy(rows.at[b], acc.at[dstv.at[b]], add=True)

        @pl.when(j + NBUF < K)
        def _():
            fetch(j + NBUF, b)

    plsc.subcore_barrier()
    pltpu.sync_copy(acc.at[pl.ds(s * RPS, RPS)],
                    out_hbm.at[c, pl.ds(s * RPS, RPS)])


@functools.partial(
    pl.kernel,
    out_type=jax.ShapeDtypeStruct((NC, NP, LATENT), jnp.float32),
    mesh=_mesh,
    scratch_types=[
        pltpu.VMEM((NBUF, CH), jnp.int32),
        pltpu.VMEM((NBUF, CH), jnp.int32),
        pltpu.VMEM((NBUF, CH, LATENT), jnp.float32),
        pltpu.VMEM_SHARED((NP, LATENT), jnp.float32),
        pltpu.SemaphoreType.DMA((NBUF,)),
    ],
)
def _spmm(t_hbm, src_hbm, dst_hbm, z_hbm, out_hbm, srcv, dstv, rows, acc, sem):
    _spmm_body(t_hbm, src_hbm, dst_hbm, z_hbm, out_hbm,
               srcv, dstv, rows, acc, sem)


BLK = 1000
NB = N // BLK


def _edge_linear_kernel(ef, We, be, q_out):
    q_out[...] = (jnp.dot(ef[...], We[...], preferred_element_type=jnp.float32)
                  + be[...])


EBLK = 1264
NEB = E_PAD // EBLK


def _run_edge_linear(ef_pad, We, be1):
    return pl.pallas_call(
        _edge_linear_kernel,
        grid=(NEB,),
        in_specs=[pl.BlockSpec((EBLK, 16), lambda i: (i, 0)),
                  _const_spec((16, LATENT)), _const_spec((1, LATENT))],
        out_specs=pl.BlockSpec((EBLK, LATENT), lambda i: (i, 0)),
        out_shape=jax.ShapeDtypeStruct((E_PAD, LATENT), jnp.float32),
    )(ef_pad, We, be1)


def _msg_kernel(nf, Wn, bn, ef0, ef1, bc, Wc, t0_out, m_out):
    msg = (jnp.dot(nf[...], Wn[...], preferred_element_type=jnp.float32)
           + bn[...] + ef0[...][0] + ef1[...][0])
    m_out[...] = msg + bc[...]
    t0_out[...] = jnp.dot(jax.nn.relu(msg), Wc[...],
                          preferred_element_type=jnp.float32)


def _iter_kernel(p0, p1, m, Wc, t_out):
    cur = jax.nn.relu(p0[...][0] + p1[...][0] + m[...])
    t_out[...] = jnp.dot(cur, Wc[...], preferred_element_type=jnp.float32)


def _final_kernel(p0, p1, m, Wo, bo, gids, y_out):
    i = pl.program_id(0)
    cur = jax.nn.relu(p0[...][0] + p1[...][0] + m[...])
    a = jax.nn.relu(jnp.dot(cur, Wo[...], preferred_element_type=jnp.float32)
                    + bo[...])
    gid = gids[0, 0, :].reshape(1, BLK)
    oh = (lax.broadcasted_iota(jnp.int32, (G, BLK), 0) == gid
          ).astype(jnp.float32)
    y = jnp.dot(oh, a, preferred_element_type=jnp.float32)

    @pl.when(i == 0)
    def _():
        y_out[...] = jnp.zeros_like(y_out)

    acc = y_out[...] + y
    y_out[...] = jnp.where(i == NB - 1, jax.nn.relu(acc), acc)


def _row_spec(cols):
    return pl.BlockSpec((BLK, cols), lambda i: (i, 0))


def _part_spec(core):
    return pl.BlockSpec((1, BLK, LATENT), lambda i, core=core: (core, i, 0))


def _const_spec(shape):
    nd = len(shape)
    return pl.BlockSpec(shape, lambda i: (0,) * nd)


def _run_msg(nf, Wn, bn, ef_part, bc, Wc):
    return pl.pallas_call(
        _msg_kernel,
        grid=(NB,),
        in_specs=[
            _row_spec(LATENT), _const_spec((LATENT, LATENT)),
            _const_spec((1, LATENT)),
            _part_spec(0), _part_spec(1),
            _const_spec((1, LATENT)), _const_spec((LATENT, LATENT)),
        ],
        out_specs=[_row_spec(LATENT), _row_spec(LATENT)],
        out_shape=[jax.ShapeDtypeStruct((N, LATENT), jnp.float32),
                   jax.ShapeDtypeStruct((N, LATENT), jnp.float32)],
    )(nf, Wn, bn, ef_part, ef_part, bc, Wc)


def _run_iter(p_part, m, Wc):
    return pl.pallas_call(
        _iter_kernel,
        grid=(NB,),
        in_specs=[_part_spec(0), _part_spec(1), _row_spec(LATENT),
                  _const_spec((LATENT, LATENT))],
        out_specs=_row_spec(LATENT),
        out_shape=jax.ShapeDtypeStruct((N, LATENT), jnp.float32),
    )(p_part, p_part, m, Wc)


def _run_final(p_part, m, Wo, bo, gids3):
    return pl.pallas_call(
        _final_kernel,
        grid=(NB,),
        in_specs=[_part_spec(0), _part_spec(1), _row_spec(LATENT),
                  _const_spec((LATENT, LATENT)), _const_spec((1, LATENT)),
                  pl.BlockSpec((1, 1, BLK), lambda i: (i, 0, 0))],
        out_specs=_const_spec((G, LATENT)),
        out_shape=jax.ShapeDtypeStruct((G, LATENT), jnp.float32),
        compiler_params=pltpu.CompilerParams(
            dimension_semantics=("arbitrary",)),
    )(p_part, p_part, m, Wo, bo, gids3)


def kernel(node_feat, edge_feat, edge_index, graph_ids,
           Wn, bn, We, be, Wc, bc, Wo, bo):
    src = edge_index[0]
    dst = edge_index[1]
    pad = E_PAD - E

    ef_pad = jnp.concatenate([edge_feat, jnp.zeros((pad, 16), jnp.float32)],
                             axis=0)
    src_p = jnp.concatenate([src, jnp.zeros((pad,), jnp.int32)]
                            ).reshape(NW, K, CH)
    dst_p = jnp.concatenate([dst, jnp.full((pad,), N, jnp.int32)]
                            ).reshape(NW, K, CH)

    z128 = jnp.zeros((RPS, LATENT), jnp.float32)

    bn1 = bn.reshape(1, LATENT)
    be1 = be.reshape(1, LATENT)
    bc1 = bc.reshape(1, LATENT)
    bo1 = bo.reshape(1, LATENT)
    gids3 = graph_ids.reshape(NB, 1, BLK)

    q = _run_edge_linear(ef_pad, We, be1)
    ef_part = _edge_pool(q, dst_p, z128)
    t, m = _run_msg(node_feat, Wn, bn1, ef_part, bc1, Wc)
    for lv in range(MAX_LV):
        p_part = _spmm(t, src_p, dst_p, z128)
        if lv < MAX_LV - 1:
            t = _run_iter(p_part, m, Wc)
    return _run_final(p_part, m, Wo, bo1, gids3)


# speedup vs baseline: 1.3749x; 1.3749x over previous
import functools

import jax
import jax.numpy as jnp
from jax import lax
from jax.experimental import pallas as pl
from jax.experimental.pallas import tpu as pltpu
from jax.experimental.pallas import tpu_sc as plsc

N = 10000
E = 320000
G = 64
LATENT = 128
MAX_LV = 3

NC = 2
NS = 16
NW = NC * NS
CH = 128
K = 79
E_PAD = NW * K * CH
TOTC = E_PAD // CH
K0 = 110
K1 = 48
NP = 10112
RPS = NP // NS

NBUF = 2

_mesh = plsc.VectorSubcoreMesh(core_axis_name="c", subcore_axis_name="s")


def _edge_pool_body(ef_hbm, dst_hbm, z_hbm, out_hbm, dstv, rows, acc, sem):
    c = lax.axis_index("c")
    s = lax.axis_index("s")
    w = c * NS + s
    pltpu.sync_copy(z_hbm, acc.at[pl.ds(s * RPS, RPS)])
    plsc.subcore_barrier()

    def fetch(j, b):
        base = (w * K + j) * CH
        pltpu.sync_copy(dst_hbm.at[w * K + j], dstv.at[b])
        pltpu.async_copy(ef_hbm.at[pl.ds(base, CH)], rows.at[b], sem.at[b])

    fetch(0, 0)

    @pl.loop(0, K)
    def _chunk(j):
        b = lax.rem(j, 2)

        @pl.when(j + 1 < K)
        def _():
            fetch(j + 1, 1 - b)

        base = (w * K + j) * CH
        pltpu.make_async_copy(ef_hbm.at[pl.ds(base, CH)], rows.at[b],
                              sem.at[b]).wait()
        pltpu.sync_copy(rows.at[b], acc.at[dstv.at[b]], add=True)

    plsc.subcore_barrier()
    pltpu.sync_copy(acc.at[pl.ds(s * RPS, RPS)],
                    out_hbm.at[c, pl.ds(s * RPS, RPS)])


@functools.partial(
    pl.kernel,
    out_type=jax.ShapeDtypeStruct((NC, NP, LATENT), jnp.float32),
    mesh=_mesh,
    scratch_types=[
        pltpu.VMEM((2, CH), jnp.int32),
        pltpu.VMEM((2, CH, LATENT), jnp.float32),
        pltpu.VMEM_SHARED((NP, LATENT), jnp.float32),
        pltpu.SemaphoreType.DMA((2,)),
    ],
)
def _edge_pool(ef_hbm, dst_hbm, z_hbm, out_hbm, dstv, rows, acc, sem):
    _edge_pool_body(ef_hbm, dst_hbm, z_hbm, out_hbm, dstv, rows, acc, sem)


def _spmm_body(t_hbm, src_hbm, dst_hbm, z_hbm, out_hbm,
               srcv, dstv, rows, acc, sem):
    c = lax.axis_index("c")
    s = lax.axis_index("s")
    pltpu.sync_copy(z_hbm, acc.at[pl.ds(s * RPS, RPS)])
    plsc.subcore_barrier()

    kc = jnp.where(c == 0, K0, K1)
    base = jnp.where(c == 0, s * K0, NS * K0 + s * K1)

    def fetch(j, b):
        g = base + j
        pltpu.sync_copy(src_hbm.at[g], srcv.at[b])
        pltpu.sync_copy(dst_hbm.at[g], dstv.at[b])
        pltpu.async_copy(t_hbm.at[srcv.at[b]], rows.at[b], sem.at[b])

    for b0 in range(NBUF):
        fetch(b0, b0)

    @pl.loop(0, K0)
    def _chunk(j):
        @pl.when(j < kc)
        def _():
            b = lax.rem(j, NBUF)

            pltpu.make_async_copy(t_hbm.at[srcv.at[b]], rows.at[b],
                                  sem.at[b]).wait()
            pltpu.sync_copy(rows.at[b], acc.at[dstv.at[b]], add=True)

            @pl.when(j + NBUF < kc)
            def _():
                fetch(j + NBUF, b)

    plsc.subcore_barrier()
    pltpu.sync_copy(acc.at[pl.ds(s * RPS, RPS)],
                    out_hbm.at[c, pl.ds(s * RPS, RPS)])


@functools.partial(
    pl.kernel,
    out_type=jax.ShapeDtypeStruct((NC, NP, LATENT), jnp.float32),
    mesh=_mesh,
    scratch_types=[
        pltpu.VMEM((NBUF, CH), jnp.int32),
        pltpu.VMEM((NBUF, CH), jnp.int32),
        pltpu.VMEM((NBUF, CH, LATENT), jnp.float32),
        pltpu.VMEM_SHARED((NP, LATENT), jnp.float32),
        pltpu.SemaphoreType.DMA((NBUF,)),
    ],
)
def _spmm(t_hbm, src_hbm, dst_hbm, z_hbm, out_hbm, srcv, dstv, rows, acc, sem):
    _spmm_body(t_hbm, src_hbm, dst_hbm, z_hbm, out_hbm,
               srcv, dstv, rows, acc, sem)


BLK = 1000
NB = N // BLK


def _edge_linear_kernel(ef, We, be, q_out):
    q_out[...] = (jnp.dot(ef[...], We[...], preferred_element_type=jnp.float32)
                  + be[...])


EBLK = 2000
NEB = E // EBLK


def _run_edge_linear(ef, We, be1):
    return pl.pallas_call(
        _edge_linear_kernel,
        grid=(NEB,),
        in_specs=[pl.BlockSpec((EBLK, 16), lambda i: (i, 0)),
                  _const_spec((16, LATENT)), _const_spec((1, LATENT))],
        out_specs=pl.BlockSpec((EBLK, LATENT), lambda i: (i, 0)),
        out_shape=jax.ShapeDtypeStruct((E_PAD, LATENT), jnp.float32),
    )(ef, We, be1)


def _msg_kernel(nf, Wn, bn, ef0, ef1, bc, Wc, t0_out, m_out):
    msg = (jnp.dot(nf[...], Wn[...], preferred_element_type=jnp.float32)
           + bn[...] + ef0[...][0] + ef1[...][0])
    m_out[...] = msg + bc[...]
    t0_out[...] = jnp.dot(jax.nn.relu(msg), Wc[...],
                          preferred_element_type=jnp.float32)


def _iter_kernel(p0, p1, m, Wc, t_out):
    cur = jax.nn.relu(p0[...][0] + p1[...][0] + m[...])
    t_out[...] = jnp.dot(cur, Wc[...], preferred_element_type=jnp.float32)


def _final_kernel(p0, p1, m, Wo, bo, gids, y_out):
    i = pl.program_id(0)
    cur = jax.nn.relu(p0[...][0] + p1[...][0] + m[...])
    a = jax.nn.relu(jnp.dot(cur, Wo[...], preferred_element_type=jnp.float32)
                    + bo[...])
    gid = gids[0, 0, :].reshape(1, BLK)
    oh = (lax.broadcasted_iota(jnp.int32, (G, BLK), 0) == gid
          ).astype(jnp.float32)
    y = jnp.dot(oh, a, preferred_element_type=jnp.float32)

    @pl.when(i == 0)
    def _():
        y_out[...] = jnp.zeros_like(y_out)

    acc = y_out[...] + y
    y_out[...] = jnp.where(i == NB - 1, jax.nn.relu(acc), acc)


def _row_spec(cols):
    return pl.BlockSpec((BLK, cols), lambda i: (i, 0))


def _part_spec(core):
    return pl.BlockSpec((1, BLK, LATENT), lambda i, core=core: (core, i, 0))


def _const_spec(shape):
    nd = len(shape)
    return pl.BlockSpec(shape, lambda i: (0,) * nd)


def _run_msg(nf, Wn, bn, ef_part, bc, Wc):
    return pl.pallas_call(
        _msg_kernel,
        grid=(NB,),
        in_specs=[
            _row_spec(LATENT), _const_spec((LATENT, LATENT)),
            _const_spec((1, LATENT)),
            _part_spec(0), _part_spec(1),
            _const_spec((1, LATENT)), _const_spec((LATENT, LATENT)),
        ],
        out_specs=[_row_spec(LATENT), _row_spec(LATENT)],
        out_shape=[jax.ShapeDtypeStruct((N, LATENT), jnp.float32),
                   jax.ShapeDtypeStruct((N, LATENT), jnp.float32)],
    )(nf, Wn, bn, ef_part, ef_part, bc, Wc)


def _run_iter(p_part, m, Wc):
    return pl.pallas_call(
        _iter_kernel,
        grid=(NB,),
        in_specs=[_part_spec(0), _part_spec(1), _row_spec(LATENT),
                  _const_spec((LATENT, LATENT))],
        out_specs=_row_spec(LATENT),
        out_shape=jax.ShapeDtypeStruct((N, LATENT), jnp.float32),
    )(p_part, p_part, m, Wc)


def _run_final(p_part, m, Wo, bo, gids3):
    return pl.pallas_call(
        _final_kernel,
        grid=(NB,),
        in_specs=[_part_spec(0), _part_spec(1), _row_spec(LATENT),
                  _const_spec((LATENT, LATENT)), _const_spec((1, LATENT)),
                  pl.BlockSpec((1, 1, BLK), lambda i: (i, 0, 0))],
        out_specs=_const_spec((G, LATENT)),
        out_shape=jax.ShapeDtypeStruct((G, LATENT), jnp.float32),
        compiler_params=pltpu.CompilerParams(
            dimension_semantics=("arbitrary",)),
    )(p_part, p_part, m, Wo, bo, gids3)


def kernel(node_feat, edge_feat, edge_index, graph_ids,
           Wn, bn, We, be, Wc, bc, Wo, bo):
    src = edge_index[0]
    dst = edge_index[1]
    pad = E_PAD - E

    src_p = jnp.concatenate([src, jnp.zeros((pad,), jnp.int32)]
                            ).reshape(TOTC, CH)
    dst_p = jnp.concatenate([dst, jnp.full((pad,), N, jnp.int32)]
                            ).reshape(TOTC, CH)

    z128 = jnp.zeros((RPS, LATENT), jnp.float32)

    bn1 = bn.reshape(1, LATENT)
    be1 = be.reshape(1, LATENT)
    bc1 = bc.reshape(1, LATENT)
    bo1 = bo.reshape(1, LATENT)
    gids3 = graph_ids.reshape(NB, 1, BLK)

    q = _run_edge_linear(edge_feat, We, be1)
    ef_part = _edge_pool(q, dst_p, z128)
    t, m = _run_msg(node_feat, Wn, bn1, ef_part, bc1, Wc)
    for lv in range(MAX_LV):
        p_part = _spmm(t, src_p, dst_p, z128)
        if lv < MAX_LV - 1:
            t = _run_iter(p_part, m, Wc)
    return _run_final(p_part, m, Wo, bo1, gids3)


# speedup vs baseline: 1.5815x; 1.1503x over previous
import functools

import jax
import jax.numpy as jnp
from jax import lax
from jax.experimental import pallas as pl
from jax.experimental.pallas import tpu as pltpu
from jax.experimental.pallas import tpu_sc as plsc

N = 10000
E = 320000
G = 64
LATENT = 128
MAX_LV = 3

NC = 2
NS = 16
NW = NC * NS
CH = 128
K = 79
E_PAD = NW * K * CH
TOTC = E_PAD // CH
K0 = 114
K1 = 44
NP = 10112
RPS = NP // NS

NBUF = 2

_mesh = plsc.VectorSubcoreMesh(core_axis_name="c", subcore_axis_name="s")


def _edge_pool_body(ef_hbm, dst_hbm, z_hbm, out_hbm, dstv, rows, acc, sem):
    c = lax.axis_index("c")
    s = lax.axis_index("s")
    w = c * NS + s
    pltpu.sync_copy(z_hbm, acc.at[pl.ds(s * RPS, RPS)])
    plsc.subcore_barrier()

    def fetch(j, b):
        base = (w * K + j) * CH
        pltpu.sync_copy(dst_hbm.at[w * K + j], dstv.at[b])
        pltpu.async_copy(ef_hbm.at[pl.ds(base, CH)], rows.at[b], sem.at[b])

    fetch(0, 0)

    @pl.loop(0, K)
    def _chunk(j):
        b = lax.rem(j, 2)

        @pl.when(j + 1 < K)
        def _():
            fetch(j + 1, 1 - b)

        base = (w * K + j) * CH
        pltpu.make_async_copy(ef_hbm.at[pl.ds(base, CH)], rows.at[b],
                              sem.at[b]).wait()
        pltpu.sync_copy(rows.at[b], acc.at[dstv.at[b]], add=True)

    plsc.subcore_barrier()
    pltpu.sync_copy(acc.at[pl.ds(s * RPS, RPS)],
                    out_hbm.at[c, pl.ds(s * RPS, RPS)])


@functools.partial(
    pl.kernel,
    out_type=jax.ShapeDtypeStruct((NC, NP, LATENT), jnp.float32),
    mesh=_mesh,
    scratch_types=[
        pltpu.VMEM((2, CH), jnp.int32),
        pltpu.VMEM((2, CH, LATENT), jnp.float32),
        pltpu.VMEM_SHARED((NP, LATENT), jnp.float32),
        pltpu.SemaphoreType.DMA((2,)),
    ],
)
def _edge_pool(ef_hbm, dst_hbm, z_hbm, out_hbm, dstv, rows, acc, sem):
    _edge_pool_body(ef_hbm, dst_hbm, z_hbm, out_hbm, dstv, rows, acc, sem)


def _spmm_body(t_hbm, src_hbm, dst_hbm, z_hbm, out_hbm,
               srcv, dstv, rows, acc, sem):
    c = lax.axis_index("c")
    s = lax.axis_index("s")
    pltpu.sync_copy(z_hbm, acc.at[pl.ds(s * RPS, RPS)])
    plsc.subcore_barrier()

    kc = jnp.where(c == 0, K0, K1)
    base = jnp.where(c == 0, s * K0, NS * K0 + s * K1)

    def fetch(j, b):
        g = base + j
        pltpu.sync_copy(src_hbm.at[g], srcv.at[b])
        pltpu.sync_copy(dst_hbm.at[g], dstv.at[b])
        pltpu.async_copy(t_hbm.at[srcv.at[b]], rows.at[b], sem.at[b])

    for b0 in range(NBUF):
        fetch(b0, b0)

    @pl.loop(0, K0)
    def _chunk(j):
        @pl.when(j < kc)
        def _():
            b = lax.rem(j, NBUF)

            pltpu.make_async_copy(t_hbm.at[srcv.at[b]], rows.at[b],
                                  sem.at[b]).wait()
            pltpu.sync_copy(rows.at[b], acc.at[dstv.at[b]], add=True)

            @pl.when(j + NBUF < kc)
            def _():
                fetch(j + NBUF, b)

    plsc.subcore_barrier()
    pltpu.sync_copy(acc.at[pl.ds(s * RPS, RPS)],
                    out_hbm.at[c, pl.ds(s * RPS, RPS)])


@functools.partial(
    pl.kernel,
    out_type=jax.ShapeDtypeStruct((NC, NP, LATENT), jnp.float32),
    mesh=_mesh,
    scratch_types=[
        pltpu.VMEM((NBUF, CH), jnp.int32),
        pltpu.VMEM((NBUF, CH), jnp.int32),
        pltpu.VMEM((NBUF, CH, LATENT), jnp.float32),
        pltpu.VMEM_SHARED((NP, LATENT), jnp.float32),
        pltpu.SemaphoreType.DMA((NBUF,)),
    ],
)
def _spmm(t_hbm, src_hbm, dst_hbm, z_hbm, out_hbm, srcv, dstv, rows, acc, sem):
    _spmm_body(t_hbm, src_hbm, dst_hbm, z_hbm, out_hbm,
               srcv, dstv, rows, acc, sem)


BLK = 1000
NB = N // BLK


def _edge_linear_kernel(efT, We, be, q_out):
    q_out[...] = lax.dot_general(
        efT[...], We[...], (((0,), (0,)), ((), ())),
        preferred_element_type=jnp.float32) + be[...]


EBLK = 3200
NEB = E // EBLK


def _run_edge_linear(efT, We, be1):
    return pl.pallas_call(
        _edge_linear_kernel,
        grid=(NEB,),
        in_specs=[pl.BlockSpec((16, EBLK), lambda i: (0, i)),
                  _const_spec((16, LATENT)), _const_spec((1, LATENT))],
        out_specs=pl.BlockSpec((EBLK, LATENT), lambda i: (i, 0)),
        out_shape=jax.ShapeDtypeStruct((E_PAD, LATENT), jnp.float32),
    )(efT, We, be1)


def _msg_kernel(nf, Wn, bn, ef0, ef1, bc, Wc, t0_out, m_out):
    msg = (jnp.dot(nf[...], Wn[...], preferred_element_type=jnp.float32)
           + bn[...] + ef0[...][0] + ef1[...][0])
    m_out[...] = msg + bc[...]
    t0_out[...] = jnp.dot(jax.nn.relu(msg), Wc[...],
                          preferred_element_type=jnp.float32)


def _iter_kernel(p0, p1, m, Wc, t_out):
    cur = jax.nn.relu(p0[...][0] + p1[...][0] + m[...])
    t_out[...] = jnp.dot(cur, Wc[...], preferred_element_type=jnp.float32)


def _final_kernel(p0, p1, m, Wo, bo, gids, y_out):
    i = pl.program_id(0)
    cur = jax.nn.relu(p0[...][0] + p1[...][0] + m[...])
    a = jax.nn.relu(jnp.dot(cur, Wo[...], preferred_element_type=jnp.float32)
                    + bo[...])
    gid = gids[0, 0, :].reshape(1, BLK)
    oh = (lax.broadcasted_iota(jnp.int32, (G, BLK), 0) == gid
          ).astype(jnp.float32)
    y = jnp.dot(oh, a, preferred_element_type=jnp.float32)

    @pl.when(i == 0)
    def _():
        y_out[...] = jnp.zeros_like(y_out)

    acc = y_out[...] + y
    y_out[...] = jnp.where(i == NB - 1, jax.nn.relu(acc), acc)


def _row_spec(cols):
    return pl.BlockSpec((BLK, cols), lambda i: (i, 0))


def _part_spec(core):
    return pl.BlockSpec((1, BLK, LATENT), lambda i, core=core: (core, i, 0))


def _const_spec(shape):
    nd = len(shape)
    return pl.BlockSpec(shape, lambda i: (0,) * nd)


def _run_msg(nf, Wn, bn, ef_part, bc, Wc):
    return pl.pallas_call(
        _msg_kernel,
        grid=(NB,),
        in_specs=[
            _row_spec(LATENT), _const_spec((LATENT, LATENT)),
            _const_spec((1, LATENT)),
            _part_spec(0), _part_spec(1),
            _const_spec((1, LATENT)), _const_spec((LATENT, LATENT)),
        ],
        out_specs=[_row_spec(LATENT), _row_spec(LATENT)],
        out_shape=[jax.ShapeDtypeStruct((N, LATENT), jnp.float32),
                   jax.ShapeDtypeStruct((N, LATENT), jnp.float32)],
    )(nf, Wn, bn, ef_part, ef_part, bc, Wc)


def _run_iter(p_part, m, Wc):
    return pl.pallas_call(
        _iter_kernel,
        grid=(NB,),
        in_specs=[_part_spec(0), _part_spec(1), _row_spec(LATENT),
                  _const_spec((LATENT, LATENT))],
        out_specs=_row_spec(LATENT),
        out_shape=jax.ShapeDtypeStruct((N, LATENT), jnp.float32),
    )(p_part, p_part, m, Wc)


def _run_final(p_part, m, Wo, bo, gids3):
    return pl.pallas_call(
        _final_kernel,
        grid=(NB,),
        in_specs=[_part_spec(0), _part_spec(1), _row_spec(LATENT),
                  _const_spec((LATENT, LATENT)), _const_spec((1, LATENT)),
                  pl.BlockSpec((1, 1, BLK), lambda i: (i, 0, 0))],
        out_specs=_const_spec((G, LATENT)),
        out_shape=jax.ShapeDtypeStruct((G, LATENT), jnp.float32),
        compiler_params=pltpu.CompilerParams(
            dimension_semantics=("arbitrary",)),
    )(p_part, p_part, m, Wo, bo, gids3)


def kernel(node_feat, edge_feat, edge_index, graph_ids,
           Wn, bn, We, be, Wc, bc, Wo, bo):
    src = edge_index[0]
    dst = edge_index[1]
    pad = E_PAD - E

    src_p = jnp.concatenate([src, jnp.zeros((pad,), jnp.int32)]
                            ).reshape(TOTC, CH)
    dst_p = jnp.concatenate([dst, jnp.full((pad,), N, jnp.int32)]
                            ).reshape(TOTC, CH)

    z128 = jnp.zeros((RPS, LATENT), jnp.float32)

    bn1 = bn.reshape(1, LATENT)
    be1 = be.reshape(1, LATENT)
    bc1 = bc.reshape(1, LATENT)
    bo1 = bo.reshape(1, LATENT)
    gids3 = graph_ids.reshape(NB, 1, BLK)

    q = _run_edge_linear(edge_feat.T, We, be1)
    ef_part = _edge_pool(q, dst_p, z128)
    t, m = _run_msg(node_feat, Wn, bn1, ef_part, bc1, Wc)
    for lv in range(MAX_LV):
        p_part = _spmm(t, src_p, dst_p, z128)
        if lv < MAX_LV - 1:
            t = _run_iter(p_part, m, Wc)
    return _run_final(p_part, m, Wo, bo1, gids3)


# speedup vs baseline: 2.1389x; 1.3524x over previous
import functools

import jax
import jax.numpy as jnp
from jax import lax
from jax.experimental import pallas as pl
from jax.experimental.pallas import tpu as pltpu
from jax.experimental.pallas import tpu_sc as plsc

N = 10000
E = 320000
G = 64
LATENT = 128
MAX_LV = 3

NC = 2
NS = 16
NW = NC * NS
CH = 128
K = 79
E_PAD = NW * K * CH
TOTC = E_PAD // CH
K0 = 79
K1 = 79
NP = 10112
RPS = NP // NS

NBUF = 2

_mesh = plsc.VectorSubcoreMesh(core_axis_name="c", subcore_axis_name="s")


def _edge_pool_body(ef_hbm, dst_hbm, z_hbm, out_hbm, dstv, rows, acc, sem):
    c = lax.axis_index("c")
    s = lax.axis_index("s")
    w = c * NS + s
    pltpu.sync_copy(z_hbm, acc.at[pl.ds(s * RPS, RPS)])
    plsc.subcore_barrier()

    def fetch(j, b):
        base = (w * K + j) * CH
        pltpu.sync_copy(dst_hbm.at[w * K + j], dstv.at[b])
        pltpu.async_copy(ef_hbm.at[pl.ds(base, CH)], rows.at[b], sem.at[b])

    fetch(0, 0)

    @pl.loop(0, K)
    def _chunk(j):
        b = lax.rem(j, 2)

        @pl.when(j + 1 < K)
        def _():
            fetch(j + 1, 1 - b)

        base = (w * K + j) * CH
        pltpu.make_async_copy(ef_hbm.at[pl.ds(base, CH)], rows.at[b],
                              sem.at[b]).wait()
        pltpu.sync_copy(rows.at[b], acc.at[dstv.at[b]], add=True)

    plsc.subcore_barrier()
    pltpu.sync_copy(acc.at[pl.ds(s * RPS, RPS)],
                    out_hbm.at[c, pl.ds(s * RPS, RPS)])


@functools.partial(
    pl.kernel,
    out_type=jax.ShapeDtypeStruct((NC, NP, LATENT), jnp.float32),
    mesh=_mesh,
    scratch_types=[
        pltpu.VMEM((2, CH), jnp.int32),
        pltpu.VMEM((2, CH, LATENT), jnp.float32),
        pltpu.VMEM_SHARED((NP, LATENT), jnp.float32),
        pltpu.SemaphoreType.DMA((2,)),
    ],
)
def _edge_pool(ef_hbm, dst_hbm, z_hbm, out_hbm, dstv, rows, acc, sem):
    _edge_pool_body(ef_hbm, dst_hbm, z_hbm, out_hbm, dstv, rows, acc, sem)


def _spmm_body(t_hbm, src_hbm, dst_hbm, z_hbm, out_hbm,
               srcv, dstv, rows, acc, sem):
    c = lax.axis_index("c")
    s = lax.axis_index("s")
    pltpu.sync_copy(z_hbm, acc.at[pl.ds(s * RPS, RPS)])
    plsc.subcore_barrier()

    kc = jnp.where(c == 0, K0, K1)
    base = jnp.where(c == 0, s * K0, NS * K0 + s * K1)

    def fetch(j, b):
        g = base + j
        pltpu.sync_copy(src_hbm.at[g], srcv.at[b])
        pltpu.sync_copy(dst_hbm.at[g], dstv.at[b])
        pltpu.async_copy(t_hbm.at[srcv.at[b]], rows.at[b], sem.at[b])

    for b0 in range(NBUF):
        fetch(b0, b0)

    @pl.loop(0, K0)
    def _chunk(j):
        @pl.when(j < kc)
        def _():
            b = lax.rem(j, NBUF)

            pltpu.make_async_copy(t_hbm.at[srcv.at[b]], rows.at[b],
                                  sem.at[b]).wait()
            pltpu.sync_copy(rows.at[b], acc.at[dstv.at[b]], add=True)

            @pl.when(j + NBUF < kc)
            def _():
                fetch(j + NBUF, b)

    plsc.subcore_barrier()
    pltpu.sync_copy(acc.at[pl.ds(s * RPS, RPS)],
                    out_hbm.at[c, pl.ds(s * RPS, RPS)])


@functools.partial(
    pl.kernel,
    out_type=jax.ShapeDtypeStruct((NC, NP, LATENT), jnp.float32),
    mesh=_mesh,
    scratch_types=[
        pltpu.VMEM((NBUF, CH), jnp.int32),
        pltpu.VMEM((NBUF, CH), jnp.int32),
        pltpu.VMEM((NBUF, CH, LATENT), jnp.float32),
        pltpu.VMEM_SHARED((NP, LATENT), jnp.float32),
        pltpu.SemaphoreType.DMA((NBUF,)),
    ],
)
def _spmm(t_hbm, src_hbm, dst_hbm, z_hbm, out_hbm, srcv, dstv, rows, acc, sem):
    _spmm_body(t_hbm, src_hbm, dst_hbm, z_hbm, out_hbm,
               srcv, dstv, rows, acc, sem)


BLK = 1000
NB = N // BLK


def _edge_linear_kernel(efT, We, be, q_out):
    q_out[...] = lax.dot_general(
        efT[...], We[...], (((0,), (0,)), ((), ())),
        preferred_element_type=jnp.float32) + be[...]


EBLK = 3200
NEB = E // EBLK


def _run_edge_linear(efT, We, be1):
    return pl.pallas_call(
        _edge_linear_kernel,
        grid=(NEB,),
        in_specs=[pl.BlockSpec((16, EBLK), lambda i: (0, i)),
                  _const_spec((16, LATENT)), _const_spec((1, LATENT))],
        out_specs=pl.BlockSpec((EBLK, LATENT), lambda i: (i, 0)),
        out_shape=jax.ShapeDtypeStruct((E_PAD, LATENT), jnp.float32),
    )(efT, We, be1)


def _msg_kernel(nf, Wn, bn, ef0, ef1, bc, Wc, t0_out, m_out):
    msg = (jnp.dot(nf[...], Wn[...], preferred_element_type=jnp.float32)
           + bn[...] + ef0[...][0] + ef1[...][0])
    m_out[...] = msg + bc[...]
    t0_out[...] = jnp.dot(jax.nn.relu(msg), Wc[...],
                          preferred_element_type=jnp.float32)


def _iter_kernel(p0, p1, m, Wc, t_out):
    cur = jax.nn.relu(p0[...][0] + p1[...][0] + m[...])
    t_out[...] = jnp.dot(cur, Wc[...], preferred_element_type=jnp.float32)


def _final_kernel(p0, p1, m, Wo, bo, gids, y_out):
    i = pl.program_id(0)
    cur = jax.nn.relu(p0[...][0] + p1[...][0] + m[...])
    a = jax.nn.relu(jnp.dot(cur, Wo[...], preferred_element_type=jnp.float32)
                    + bo[...])
    gid = gids[0, 0, :].reshape(1, BLK)
    oh = (lax.broadcasted_iota(jnp.int32, (G, BLK), 0) == gid
          ).astype(jnp.float32)
    y = jnp.dot(oh, a, preferred_element_type=jnp.float32)

    @pl.when(i == 0)
    def _():
        y_out[...] = jnp.zeros_like(y_out)

    acc = y_out[...] + y
    y_out[...] = jnp.where(i == NB - 1, jax.nn.relu(acc), acc)


def _row_spec(cols):
    return pl.BlockSpec((BLK, cols), lambda i: (i, 0))


def _part_spec(core):
    return pl.BlockSpec((1, BLK, LATENT), lambda i, core=core: (core, i, 0))


def _const_spec(shape):
    nd = len(shape)
    return pl.BlockSpec(shape, lambda i: (0,) * nd)


def _run_msg(nf, Wn, bn, ef_part, bc, Wc):
    return pl.pallas_call(
        _msg_kernel,
        grid=(NB,),
        in_specs=[
            _row_spec(LATENT), _const_spec((LATENT, LATENT)),
            _const_spec((1, LATENT)),
            _part_spec(0), _part_spec(1),
            _const_spec((1, LATENT)), _const_spec((LATENT, LATENT)),
        ],
        out_specs=[_row_spec(LATENT), _row_spec(LATENT)],
        out_shape=[jax.ShapeDtypeStruct((N, LATENT), jnp.float32),
                   jax.ShapeDtypeStruct((N, LATENT), jnp.float32)],
    )(nf, Wn, bn, ef_part, ef_part, bc, Wc)


def _run_iter(p_part, m, Wc):
    return pl.pallas_call(
        _iter_kernel,
        grid=(NB,),
        in_specs=[_part_spec(0), _part_spec(1), _row_spec(LATENT),
                  _const_spec((LATENT, LATENT))],
        out_specs=_row_spec(LATENT),
        out_shape=jax.ShapeDtypeStruct((N, LATENT), jnp.float32),
    )(p_part, p_part, m, Wc)


def _run_final(p_part, m, Wo, bo, gids3):
    return pl.pallas_call(
        _final_kernel,
        grid=(NB,),
        in_specs=[_part_spec(0), _part_spec(1), _row_spec(LATENT),
                  _const_spec((LATENT, LATENT)), _const_spec((1, LATENT)),
                  pl.BlockSpec((1, 1, BLK), lambda i: (i, 0, 0))],
        out_specs=_const_spec((G, LATENT)),
        out_shape=jax.ShapeDtypeStruct((G, LATENT), jnp.float32),
        compiler_params=pltpu.CompilerParams(
            dimension_semantics=("arbitrary",)),
    )(p_part, p_part, m, Wo, bo, gids3)


def kernel(node_feat, edge_feat, edge_index, graph_ids,
           Wn, bn, We, be, Wc, bc, Wo, bo):
    src = edge_index[0]
    dst = edge_index[1]
    pad = E_PAD - E

    src_p = jnp.concatenate([src, jnp.arange(pad, dtype=jnp.int32)]
                            ).reshape(TOTC, CH)
    dst_p = jnp.concatenate([dst, jnp.full((pad,), N, jnp.int32)]
                            ).reshape(TOTC, CH)

    z128 = jnp.zeros((RPS, LATENT), jnp.float32)

    bn1 = bn.reshape(1, LATENT)
    be1 = be.reshape(1, LATENT)
    bc1 = bc.reshape(1, LATENT)
    bo1 = bo.reshape(1, LATENT)
    gids3 = graph_ids.reshape(NB, 1, BLK)

    q = _run_edge_linear(edge_feat.T, We, be1)
    ef_part = _edge_pool(q, dst_p, z128)
    t, m = _run_msg(node_feat, Wn, bn1, ef_part, bc1, Wc)
    for lv in range(MAX_LV):
        p_part = _spmm(t, src_p, dst_p, z128)
        if lv < MAX_LV - 1:
            t = _run_iter(p_part, m, Wc)
    return _run_final(p_part, m, Wo, bo1, gids3)
